# Initial kernel scaffold; baseline (speedup 1.0000x reference)
import functools
import jax
import jax.numpy as jnp
from jax import lax
from jax.experimental import pallas as pl
from jax.experimental.pallas import tpu as pltpu
from jax.experimental.pallas import tpu_sc as plsc

NC = 2
NS = 16
NW = NC * NS
LANES = 16
CHUNK = 128


def _mesh():
    return plsc.VectorSubcoreMesh(
        core_axis_name="c", subcore_axis_name="s", num_cores=NC, num_subcores=NS
    )


def _deg_body(n_acc, tpc, dstb_hbm, out_hbm, dst_v, deg_v):
    c = lax.axis_index("c")
    s = lax.axis_index("s")
    wid = c * NS + s
    pltpu.sync_copy(dstb_hbm.at[wid], dst_v)
    zero16 = jnp.zeros((LANES,), jnp.float32)
    ones16 = jnp.ones((LANES,), jnp.float32)

    def zbody(i, carry):
        deg_v[pl.ds(i * LANES, LANES)] = zero16
        return carry

    lax.fori_loop(0, n_acc // LANES, zbody, 0)

    def ebody(j, carry):
        for k in range(CHUNK // LANES):
            idx = dst_v[j, pl.ds(k * LANES, LANES)]
            plsc.addupdate_scatter(deg_v, [idx], ones16)
        return carry

    lax.fori_loop(0, tpc, ebody, 0)
    pltpu.sync_copy(deg_v, out_hbm.at[wid])


def _deg_call(dstb, n_acc, tpc):
    body = functools.partial(_deg_body, n_acc, tpc)
    return pl.kernel(
        body,
        out_type=jax.ShapeDtypeStruct((NW, n_acc), jnp.float32),
        mesh=_mesh(),
        scratch_types=[
            pltpu.VMEM((tpc, CHUNK), jnp.int32),
            pltpu.VMEM((n_acc,), jnp.float32),
        ],
    )(dstb)


def _mp_body(n_acc, tpc, xs_hbm, srcb_hbm, dstb_hbm, z_hbm, out_hbm,
             src_v, dst_v, rows_v, acc_sh):
    c = lax.axis_index("c")
    s = lax.axis_index("s")
    wid = c * NS + s
    rp = n_acc // NS
    pltpu.sync_copy(z_hbm, acc_sh.at[pl.ds(s * rp, rp)])
    pltpu.sync_copy(srcb_hbm.at[wid], src_v)
    pltpu.sync_copy(dstb_hbm.at[wid], dst_v)
    plsc.subcore_barrier()

    def body(j, carry):
        pltpu.sync_copy(xs_hbm.at[src_v.at[j]], rows_v)
        pltpu.sync_copy(rows_v, acc_sh.at[dst_v.at[j]], add=True)
        return carry

    lax.fori_loop(0, tpc, body, 0)
    plsc.subcore_barrier()
    pltpu.sync_copy(acc_sh.at[pl.ds(s * rp, rp)], out_hbm.at[c, pl.ds(s * rp, rp)])


def _mp_call(xs, srcb, dstb, zeros_blk, n_acc, tpc):
    h = xs.shape[1]
    body = functools.partial(_mp_body, n_acc, tpc)
    return pl.kernel(
        body,
        out_type=jax.ShapeDtypeStruct((NC, n_acc, h), jnp.float32),
        mesh=_mesh(),
        scratch_types=[
            pltpu.VMEM((tpc, CHUNK), jnp.int32),
            pltpu.VMEM((tpc, CHUNK), jnp.int32),
            pltpu.VMEM((CHUNK, h), jnp.float32),
            pltpu.VMEM_SHARED((n_acc, h), jnp.float32),
        ],
    )(xs, srcb, dstb, zeros_blk)


def _tc_a_body(x_ref, degp_ref, w1_ref, xs_ref, dinv_ref):
    deg = jnp.sum(degp_ref[...], axis=0) + 1.0
    dinv = lax.rsqrt(deg)
    xw = jnp.dot(x_ref[...], w1_ref[...], preferred_element_type=jnp.float32)
    xs_ref[...] = xw * dinv[:, None]
    dinv_ref[...] = dinv


def _tc_a(x, deg_parts, w1, bn):
    n, d = x.shape
    h = w1.shape[1]
    grid = n // bn
    return pl.pallas_call(
        _tc_a_body,
        grid=(grid,),
        in_specs=[
            pl.BlockSpec((bn, d), lambda i: (i, 0)),
            pl.BlockSpec((NW, bn), lambda i: (0, i)),
            pl.BlockSpec((d, h), lambda i: (0, 0)),
        ],
        out_specs=[
            pl.BlockSpec((bn, h), lambda i: (i, 0)),
            pl.BlockSpec((bn,), lambda i: (i,)),
        ],
        out_shape=[
            jax.ShapeDtypeStruct((n, h), jnp.float32),
            jax.ShapeDtypeStruct((n,), jnp.float32),
        ],
    )(x, deg_parts, w1)


def _tc_b_body(acc_ref, xs_ref, dinv_ref, b1_ref, w2_ref, xs2_ref):
    dinv = dinv_ref[...]
    pre = (acc_ref[0] + acc_ref[1] + xs_ref[...]) * dinv[:, None] + b1_ref[...][None, :]
    h1 = jnp.maximum(pre, 0.0)
    xw2 = jnp.dot(h1, w2_ref[...], preferred_element_type=jnp.float32)
    xs2_ref[...] = xw2 * dinv[:, None]


def _tc_b(acc1, xs1, dinv, b1, w2, bn):
    n, h = xs1.shape
    grid = n // bn
    return pl.pallas_call(
        _tc_b_body,
        grid=(grid,),
        in_specs=[
            pl.BlockSpec((NC, bn, h), lambda i: (0, i, 0)),
            pl.BlockSpec((bn, h), lambda i: (i, 0)),
            pl.BlockSpec((bn,), lambda i: (i,)),
            pl.BlockSpec((h,), lambda i: (0,)),
            pl.BlockSpec((h, h), lambda i: (0, 0)),
        ],
        out_specs=pl.BlockSpec((bn, h), lambda i: (i, 0)),
        out_shape=jax.ShapeDtypeStruct((n, h), jnp.float32),
    )(acc1, xs1, dinv, b1, w2)


def _tc_c_body(g, acc_ref, xs_ref, dinv_ref, b2_ref, batch_ref, wlin_ref,
               blin_ref, out_ref, pooled_ref, cnt_ref):
    i = pl.program_id(0)
    nb = pl.num_programs(0)
    dinv = dinv_ref[...]
    pre = (acc_ref[0] + acc_ref[1] + xs_ref[...]) * dinv[:, None] + b2_ref[...][None, :]
    h2 = jnp.maximum(pre, 0.0)
    oh = (batch_ref[...][:, None] ==
          lax.broadcasted_iota(jnp.int32, (1, g), 1)).astype(jnp.float32)

    @pl.when(i == 0)
    def _():
        pooled_ref[...] = jnp.zeros_like(pooled_ref)
        cnt_ref[...] = jnp.zeros_like(cnt_ref)

    pooled_ref[...] += lax.dot_general(
        oh, h2, dimension_numbers=(((0,), (0,)), ((), ())),
        preferred_element_type=jnp.float32)
    cnt_ref[...] += jnp.sum(oh, axis=0)

    @pl.when(i == nb - 1)
    def _():
        mean = pooled_ref[...] / jnp.maximum(cnt_ref[...], 1.0)[:, None]
        out_ref[...] = jnp.dot(mean, wlin_ref[...],
                               preferred_element_type=jnp.float32) + blin_ref[...][None, :]


def _tc_c(acc2, xs2, dinv, b2, batch, wlin, blin, bn):
    n, h = xs2.shape
    g_graphs = 64
    c_cls = wlin.shape[1]
    grid = n // bn
    body = functools.partial(_tc_c_body, g_graphs)
    return pl.pallas_call(
        body,
        grid=(grid,),
        in_specs=[
            pl.BlockSpec((NC, bn, h), lambda i: (0, i, 0)),
            pl.BlockSpec((bn, h), lambda i: (i, 0)),
            pl.BlockSpec((bn,), lambda i: (i,)),
            pl.BlockSpec((h,), lambda i: (0,)),
            pl.BlockSpec((bn,), lambda i: (i,)),
            pl.BlockSpec((h, c_cls), lambda i: (0, 0)),
            pl.BlockSpec((c_cls,), lambda i: (0,)),
        ],
        out_specs=pl.BlockSpec((g_graphs, c_cls), lambda i: (0, 0)),
        out_shape=jax.ShapeDtypeStruct((g_graphs, c_cls), jnp.float32),
        scratch_shapes=[
            pltpu.VMEM((g_graphs, h), jnp.float32),
            pltpu.VMEM((g_graphs,), jnp.float32),
        ],
    )(acc2, xs2, dinv, b2, batch, wlin, blin)


def kernel(x, edge_index, batch, W1, b1, W2, b2, Wlin, blin):
    n, d = x.shape
    h = W1.shape[1]
    e = edge_index.shape[1]

    n_acc = (n + NW) // NW * NW
    epg = NW * CHUNK
    tpc = (e + epg - 1) // epg
    e_pad = tpc * epg

    src = edge_index[0]
    dst = edge_index[1]
    srcb = jnp.pad(src, (0, e_pad - e)).reshape(NW, tpc, CHUNK)
    dstb = jnp.pad(dst, (0, e_pad - e), constant_values=n).reshape(NW, tpc, CHUNK)
    zeros_blk = jnp.zeros((n_acc // NS, h), jnp.float32)

    deg_parts = _deg_call(dstb, n_acc, tpc)[:, :n]

    bn = 2500
    xs1, dinv = _tc_a(x, deg_parts, W1, bn)
    acc1 = _mp_call(xs1, srcb, dstb, zeros_blk, n_acc, tpc)[:, :n]
    xs2 = _tc_b(acc1, xs1, dinv, b1, W2, bn)
    acc2 = _mp_call(xs2, srcb, dstb, zeros_blk, n_acc, tpc)[:, :n]
    return _tc_c(acc2, xs2, dinv, b2, batch, Wlin, blin, bn)


# speedup vs baseline: 12.0130x; 12.0130x over previous
import functools
import jax
import jax.numpy as jnp
from jax import lax
from jax.experimental import pallas as pl
from jax.experimental.pallas import tpu as pltpu
from jax.experimental.pallas import tpu_sc as plsc

NC = 2
NS = 16
NW = NC * NS
LANES = 16
CHUNK = 128


def _mesh():
    return plsc.VectorSubcoreMesh(
        core_axis_name="c", subcore_axis_name="s", num_cores=NC, num_subcores=NS
    )


DW = 128


def _deg_body(n_acc, tpc, dstb_hbm, ones_hbm, z_hbm, out_hbm,
              dst_v, ones_v, acc_sh):
    c = lax.axis_index("c")
    s = lax.axis_index("s")
    wid = c * NS + s
    rp = n_acc // NS
    pltpu.sync_copy(z_hbm, acc_sh.at[pl.ds(s * rp, rp)])
    pltpu.sync_copy(dstb_hbm.at[wid], dst_v)
    pltpu.sync_copy(ones_hbm, ones_v)
    plsc.subcore_barrier()

    def body(j, carry):
        pltpu.sync_copy(ones_v, acc_sh.at[dst_v.at[j]], add=True)
        return carry

    lax.fori_loop(0, tpc, body, 0)
    plsc.subcore_barrier()
    pltpu.sync_copy(acc_sh.at[pl.ds(s * rp, rp)], out_hbm.at[c, pl.ds(s * rp, rp)])


def _deg_call(dstb, n_acc, tpc):
    body = functools.partial(_deg_body, n_acc, tpc)
    ones = jnp.ones((CHUNK, DW), jnp.float32)
    zb = jnp.zeros((n_acc // NS, DW), jnp.float32)
    return pl.kernel(
        body,
        out_type=jax.ShapeDtypeStruct((NC, n_acc, DW), jnp.float32),
        mesh=_mesh(),
        scratch_types=[
            pltpu.VMEM((tpc, CHUNK), jnp.int32),
            pltpu.VMEM((CHUNK, DW), jnp.float32),
            pltpu.VMEM_SHARED((n_acc, DW), jnp.float32),
        ],
    )(dstb, ones, zb)


def _mp_body(n_acc, tpc, xs_hbm, srcb_hbm, dstb_hbm, z_hbm, out_hbm,
             src_v, dst_v, rows_v, acc_sh):
    c = lax.axis_index("c")
    s = lax.axis_index("s")
    wid = c * NS + s
    rp = n_acc // NS
    pltpu.sync_copy(z_hbm, acc_sh.at[pl.ds(s * rp, rp)])
    pltpu.sync_copy(srcb_hbm.at[wid], src_v)
    pltpu.sync_copy(dstb_hbm.at[wid], dst_v)
    plsc.subcore_barrier()

    def body(j, carry):
        pltpu.sync_copy(xs_hbm.at[src_v.at[j]], rows_v)
        pltpu.sync_copy(rows_v, acc_sh.at[dst_v.at[j]], add=True)
        return carry

    lax.fori_loop(0, tpc, body, 0)
    plsc.subcore_barrier()
    pltpu.sync_copy(acc_sh.at[pl.ds(s * rp, rp)], out_hbm.at[c, pl.ds(s * rp, rp)])


def _mp_call(xs, srcb, dstb, zeros_blk, n_acc, tpc):
    h = xs.shape[1]
    body = functools.partial(_mp_body, n_acc, tpc)
    return pl.kernel(
        body,
        out_type=jax.ShapeDtypeStruct((NC, n_acc, h), jnp.float32),
        mesh=_mesh(),
        scratch_types=[
            pltpu.VMEM((tpc, CHUNK), jnp.int32),
            pltpu.VMEM((tpc, CHUNK), jnp.int32),
            pltpu.VMEM((CHUNK, h), jnp.float32),
            pltpu.VMEM_SHARED((n_acc, h), jnp.float32),
        ],
    )(xs, srcb, dstb, zeros_blk)


def _tc_dinv_body(degp_ref, dinv_ref):
    deg = jnp.sum(degp_ref[...], axis=(0, 2)) * (1.0 / DW) + 1.0
    dinv_ref[...] = lax.rsqrt(deg)[:, None]


def _tc_dinv(deg_parts):
    n_acc = deg_parts.shape[1]
    return pl.pallas_call(
        _tc_dinv_body,
        out_shape=jax.ShapeDtypeStruct((n_acc, 1), jnp.float32),
    )(deg_parts)


def _tc_a_body(x_ref, dinv_ref, w1_ref, xs_ref):
    xw = jnp.dot(x_ref[...], w1_ref[...], preferred_element_type=jnp.float32)
    xs_ref[...] = xw * dinv_ref[...]


def _tc_a(x, dinv, w1, bn):
    n, d = x.shape
    h = w1.shape[1]
    grid = n // bn
    return pl.pallas_call(
        _tc_a_body,
        grid=(grid,),
        in_specs=[
            pl.BlockSpec((bn, d), lambda i: (i, 0)),
            pl.BlockSpec((bn, 1), lambda i: (i, 0)),
            pl.BlockSpec((d, h), lambda i: (0, 0)),
        ],
        out_specs=pl.BlockSpec((bn, h), lambda i: (i, 0)),
        out_shape=jax.ShapeDtypeStruct((n, h), jnp.float32),
    )(x, dinv, w1)


def _tc_b_body(acc_ref, xs_ref, dinv_ref, b1_ref, w2_ref, xs2_ref):
    dinv = dinv_ref[...]
    pre = (acc_ref[0] + acc_ref[1] + xs_ref[...]) * dinv + b1_ref[...][None, :]
    h1 = jnp.maximum(pre, 0.0)
    xw2 = jnp.dot(h1, w2_ref[...], preferred_element_type=jnp.float32)
    xs2_ref[...] = xw2 * dinv


def _tc_b(acc1, xs1, dinv, b1, w2, bn):
    n, h = xs1.shape
    grid = n // bn
    return pl.pallas_call(
        _tc_b_body,
        grid=(grid,),
        in_specs=[
            pl.BlockSpec((NC, bn, h), lambda i: (0, i, 0)),
            pl.BlockSpec((bn, h), lambda i: (i, 0)),
            pl.BlockSpec((bn, 1), lambda i: (i, 0)),
            pl.BlockSpec((h,), lambda i: (0,)),
            pl.BlockSpec((h, h), lambda i: (0, 0)),
        ],
        out_specs=pl.BlockSpec((bn, h), lambda i: (i, 0)),
        out_shape=jax.ShapeDtypeStruct((n, h), jnp.float32),
    )(acc1, xs1, dinv, b1, w2)


def _tc_c_body(g, acc_ref, xs_ref, dinv_ref, b2_ref, batch_ref, wlin_ref,
               blin_ref, out_ref, pooled_ref, cnt_ref):
    i = pl.program_id(0)
    nb = pl.num_programs(0)
    bn = xs_ref.shape[0]
    pre = (acc_ref[0] + acc_ref[1] + xs_ref[...]) * dinv_ref[...] + b2_ref[...][None, :]
    h2 = jnp.maximum(pre, 0.0)
    oh = (batch_ref[...] ==
          lax.broadcasted_iota(jnp.int32, (1, g), 1)).astype(jnp.float32)

    @pl.when(i == 0)
    def _():
        pooled_ref[...] = jnp.zeros_like(pooled_ref)
        cnt_ref[...] = jnp.zeros_like(cnt_ref)

    pooled_ref[...] += lax.dot_general(
        oh, h2, dimension_numbers=(((0,), (0,)), ((), ())),
        preferred_element_type=jnp.float32)
    cnt_ref[...] += lax.dot_general(
        oh, jnp.ones((bn, 1), jnp.float32),
        dimension_numbers=(((0,), (0,)), ((), ())),
        preferred_element_type=jnp.float32)

    @pl.when(i == nb - 1)
    def _():
        mean = pooled_ref[...] / jnp.maximum(cnt_ref[...], 1.0)
        out_ref[...] = jnp.dot(mean, wlin_ref[...],
                               preferred_element_type=jnp.float32) + blin_ref[...][None, :]


def _tc_c(acc2, xs2, dinv, b2, batch_col, wlin, blin, bn):
    n, h = xs2.shape
    g_graphs = 64
    c_cls = wlin.shape[1]
    grid = n // bn
    body = functools.partial(_tc_c_body, g_graphs)
    return pl.pallas_call(
        body,
        grid=(grid,),
        in_specs=[
            pl.BlockSpec((NC, bn, h), lambda i: (0, i, 0)),
            pl.BlockSpec((bn, h), lambda i: (i, 0)),
            pl.BlockSpec((bn, 1), lambda i: (i, 0)),
            pl.BlockSpec((h,), lambda i: (0,)),
            pl.BlockSpec((bn, 1), lambda i: (i, 0)),
            pl.BlockSpec((h, c_cls), lambda i: (0, 0)),
            pl.BlockSpec((c_cls,), lambda i: (0,)),
        ],
        out_specs=pl.BlockSpec((g_graphs, c_cls), lambda i: (0, 0)),
        out_shape=jax.ShapeDtypeStruct((g_graphs, c_cls), jnp.float32),
        scratch_shapes=[
            pltpu.VMEM((g_graphs, h), jnp.float32),
            pltpu.VMEM((g_graphs, 1), jnp.float32),
        ],
    )(acc2, xs2, dinv, b2, batch_col, wlin, blin)


def kernel(x, edge_index, batch, W1, b1, W2, b2, Wlin, blin):
    n, d = x.shape
    h = W1.shape[1]
    e = edge_index.shape[1]

    n_acc = (n + 1 + NS * 8 - 1) // (NS * 8) * (NS * 8)
    epg = NW * CHUNK
    tpc = (e + epg - 1) // epg
    e_pad = tpc * epg

    src = edge_index[0]
    dst = edge_index[1]
    srcb = jnp.pad(src, (0, e_pad - e)).reshape(NW, tpc, CHUNK)
    dstb = jnp.pad(dst, (0, e_pad - e), constant_values=n).reshape(NW, tpc, CHUNK)
    zeros_blk = jnp.zeros((n_acc // NS, h), jnp.float32)

    deg_parts = _deg_call(dstb, n_acc, tpc)

    bn = 2000
    dinv = _tc_dinv(deg_parts)
    xs1 = _tc_a(x, dinv, W1, bn)
    acc1 = _mp_call(xs1, srcb, dstb, zeros_blk, n_acc, tpc)
    xs2 = _tc_b(acc1, xs1, dinv, b1, W2, bn)
    acc2 = _mp_call(xs2, srcb, dstb, zeros_blk, n_acc, tpc)
    return _tc_c(acc2, xs2, dinv, b2, batch[:, None], Wlin, blin, bn)
